# Initial kernel scaffold; baseline (speedup 1.0000x reference)
import functools

import jax
import jax.numpy as jnp
from jax import lax
from jax.experimental import pallas as pl
from jax.experimental.pallas import tpu as pltpu
from jax.experimental.pallas import tpu_sc as plsc

R = 8
LANE = 16
NTILES = 16
BW = 128
NCH = 8


def _sc_agg(with_counts, n_nodes, nb):
    cols = NCH * LANE + (LANE if with_counts else 0)
    acc_rows = R * n_nodes + BW
    zshare = acc_rows // NTILES
    zr = zshare // 8
    assert zr * 8 == zshare and n_nodes % NTILES == 0
    rpt = n_nodes // NTILES
    mesh = plsc.VectorSubcoreMesh(core_axis_name="c", subcore_axis_name="s")

    def body(table, gidx, sidx, out, gidx_v, sidx_v, rows_v, zbuf, ones_v,
             sem):
        cid = lax.axis_index("c")
        sid = lax.axis_index("s")
        pltpu.sync_copy(sidx.at[sid], sidx_v)

        zvec = jnp.zeros((LANE,), jnp.float32)

        def zfill(i, _):
            zbuf[i, :] = zvec
            return 0

        lax.fori_loop(0, zr, zfill, 0)

        if with_counts:
            onehot = (lax.iota(jnp.int32, LANE) == 0).astype(jnp.float32)

            def ofill(i, _):
                ones_v[i, :] = onehot
                return 0

            lax.fori_loop(0, BW, ofill, 0)

        def acc_zero(acc):
            for k in range(8):
                pltpu.sync_copy(zbuf, acc.at[pl.ds(sid * zshare + k * zr, zr), :])
            plsc.subcore_barrier()

        def copy_out(acc, c0):
            plsc.subcore_barrier()
            for r in range(R):
                pltpu.sync_copy(
                    acc.at[pl.ds(r * n_nodes + sid * rpt, rpt), :],
                    out.at[pl.ds(sid * rpt, rpt), r, pl.ds(c0, LANE)])
            plsc.subcore_barrier()

        def run_chunk(acc, c):
            acc_zero(acc)
            pltpu.sync_copy(gidx.at[c, sid], gidx_v)

            def step(j, _):
                pltpu.async_copy(table.at[gidx_v.at[j]], rows_v, sem).wait()
                pltpu.sync_copy(rows_v, acc.at[sidx_v.at[j]], add=True)
                return 0

            lax.fori_loop(0, nb, step, 0)
            copy_out(acc, c * LANE)

        def run_counts(acc):
            acc_zero(acc)

            def step(j, _):
                pltpu.sync_copy(ones_v, acc.at[sidx_v.at[j]], add=True)
                return 0

            lax.fori_loop(0, nb, step, 0)
            copy_out(acc, NCH * LANE)

        def run(acc):
            @pl.when(cid == 0)
            def _():
                for c in range(NCH // 2):
                    run_chunk(acc, c)
                if with_counts:
                    run_counts(acc)

            @pl.when(cid == 1)
            def _():
                for c in range(NCH // 2, NCH):
                    run_chunk(acc, c)

        pl.run_scoped(run, pltpu.VMEM_SHARED((acc_rows, LANE), jnp.float32))

    return pl.kernel(
        body,
        out_type=jax.ShapeDtypeStruct((n_nodes, R, cols), jnp.float32),
        mesh=mesh,
        scratch_types=[
            pltpu.VMEM((nb, BW), jnp.int32),
            pltpu.VMEM((nb, BW), jnp.int32),
            pltpu.VMEM((BW, LANE), jnp.float32),
            pltpu.VMEM((zr, LANE), jnp.float32),
            pltpu.VMEM((BW, LANE), jnp.float32),
            pltpu.SemaphoreType.DMA,
        ],
    )


def _tc_layer1(x, agg, root, W, b, bn=400):
    n = x.shape[0]

    def body(x_ref, agg_ref, root_ref, w_ref, b_ref, out_ref):
        acc = jnp.dot(x_ref[...], root_ref[...],
                      preferred_element_type=jnp.float32) + b_ref[...]
        for r in range(R):
            s = agg_ref[:, r, 0:128]
            cnt = agg_ref[:, r, 128:129]
            h = s / jnp.maximum(cnt, 1.0)
            acc = acc + jnp.dot(h, w_ref[r], preferred_element_type=jnp.float32)
        out_ref[...] = jnp.maximum(acc, 0.0)

    return pl.pallas_call(
        body,
        grid=(n // bn,),
        in_specs=[
            pl.BlockSpec((bn, 128), lambda i: (i, 0)),
            pl.BlockSpec((bn, R, 144), lambda i: (i, 0, 0)),
            pl.BlockSpec((128, 128), lambda i: (0, 0)),
            pl.BlockSpec((R, 128, 128), lambda i: (0, 0, 0)),
            pl.BlockSpec((1, 128), lambda i: (0, 0)),
        ],
        out_specs=pl.BlockSpec((bn, 128), lambda i: (i, 0)),
        out_shape=jax.ShapeDtypeStruct((n, 128), jnp.float32),
    )(x, agg, root, W, b)


def _tc_layer2(h, agg, cnt, root, W, b, lin_w, lin_b, bn=400):
    n = h.shape[0]
    d_out = lin_w.shape[1]

    def body(h_ref, agg_ref, cnt_ref, root_ref, w_ref, b_ref, lw_ref, lb_ref,
             out_ref):
        acc = jnp.dot(h_ref[...], root_ref[...],
                      preferred_element_type=jnp.float32) + b_ref[...]
        for r in range(R):
            s = agg_ref[:, r, :]
            c = cnt_ref[:, r:r + 1]
            acc = acc + jnp.dot(s / jnp.maximum(c, 1.0), w_ref[r],
                                preferred_element_type=jnp.float32)
        h2 = jnp.maximum(acc, 0.0)
        logits = jnp.dot(h2, lw_ref[...],
                         preferred_element_type=jnp.float32) + lb_ref[...]
        m = jnp.max(logits, axis=1, keepdims=True)
        lse = jnp.log(jnp.sum(jnp.exp(logits - m), axis=1, keepdims=True)) + m
        out_ref[...] = logits - lse

    return pl.pallas_call(
        body,
        grid=(n // bn,),
        in_specs=[
            pl.BlockSpec((bn, 128), lambda i: (i, 0)),
            pl.BlockSpec((bn, R, 128), lambda i: (i, 0, 0)),
            pl.BlockSpec((bn, R), lambda i: (i, 0)),
            pl.BlockSpec((128, 128), lambda i: (0, 0)),
            pl.BlockSpec((R, 128, 128), lambda i: (0, 0, 0)),
            pl.BlockSpec((1, 128), lambda i: (0, 0)),
            pl.BlockSpec((128, d_out), lambda i: (0, 0)),
            pl.BlockSpec((1, d_out), lambda i: (0, 0)),
        ],
        out_specs=pl.BlockSpec((bn, d_out), lambda i: (i, 0)),
        out_shape=jax.ShapeDtypeStruct((n, d_out), jnp.float32),
    )(h, agg, cnt, root, W, b, lin_w, lin_b)


def kernel(x, edge_index, edge_type, W1, root1, b1, W2, root2, b2, lin_w,
           lin_b):
    n, d_in = x.shape
    e = edge_index.shape[1]
    assert d_in == 128 and n % NTILES == 0

    nb = -(-e // (NTILES * BW))
    e_pad = NTILES * nb * BW

    src = edge_index[0].astype(jnp.int32)
    dst = edge_index[1].astype(jnp.int32)
    et = edge_type.astype(jnp.int32)
    ar = jnp.arange(e_pad - e, dtype=jnp.int32)
    src_p = jnp.concatenate([src, ar % n])
    sidx = jnp.concatenate([et * n + dst, R * n + (ar % BW)])
    sidx = sidx.reshape(NTILES, nb, BW)
    gidx = (src_p[None, :] * NCH + jnp.arange(NCH, dtype=jnp.int32)[:, None])
    gidx = gidx.reshape(NCH, NTILES, nb, BW)

    agg_counts = _sc_agg(True, n, nb)
    agg_plain = _sc_agg(False, n, nb)

    agg1 = agg_counts(x.reshape(n * NCH, LANE), gidx, sidx)
    h1 = _tc_layer1(x, agg1, root1, W1, b1.reshape(1, 128))
    cnt = agg1[:, :, 128]
    agg2 = agg_plain(h1.reshape(n * NCH, LANE), gidx, sidx)
    return _tc_layer2(h1, agg2, cnt, root2, W2, b2.reshape(1, 128), lin_w,
                      lin_b.reshape(1, lin_w.shape[1]))


# speedup vs baseline: 8.5891x; 8.5891x over previous
import functools

import jax
import jax.numpy as jnp
from jax import lax
from jax.experimental import pallas as pl
from jax.experimental.pallas import tpu as pltpu
from jax.experimental.pallas import tpu_sc as plsc

R = 8
LANE = 16
NTILES = 16
BW = 128
NCH = 8
SEG = 16


def _sc_agg(with_counts, n_nodes, nb):
    cols = NCH * LANE + (LANE if with_counts else 0)
    acc_rows = R * n_nodes + BW
    zshare = acc_rows // NTILES
    zr = zshare // 8
    assert zr * 8 == zshare and n_nodes % NTILES == 0
    rpt = n_nodes // NTILES
    mesh = plsc.VectorSubcoreMesh(core_axis_name="c", subcore_axis_name="s")

    nseg = nb // SEG
    assert nseg * SEG == nb

    def body(table, gidx, sidx, out, gidx_v, sidx_v, rows_v, zbuf, ones_v,
             acc, sem):
        cid = lax.axis_index("c")
        sid = lax.axis_index("s")

        def zfill(i, _):
            zbuf[i, :] = jnp.zeros((LANE,), jnp.float32)
            return 0

        lax.fori_loop(0, zr, zfill, 0)

        if with_counts:
            def ofill(i, _):
                lane = lax.iota(jnp.int32, LANE).astype(jnp.float32)
                ones_v[i, :] = jnp.maximum(1.0 - lane, 0.0)
                return 0

            lax.fori_loop(0, BW, ofill, 0)

        def acc_zero(acc):
            for k in range(8):
                pltpu.sync_copy(zbuf, acc.at[pl.ds(sid * zshare + k * zr, zr), :])
            plsc.subcore_barrier()

        def copy_out(acc, c0):
            plsc.subcore_barrier()
            for r in range(R):
                pltpu.sync_copy(
                    acc.at[pl.ds(r * n_nodes + sid * rpt, rpt), :],
                    out.at[pl.ds(sid * rpt, rpt), r, pl.ds(c0, LANE)])
            plsc.subcore_barrier()

        def run_chunk(acc, c):
            acc_zero(acc)

            def seg_step(g, _):
                pltpu.sync_copy(gidx.at[c, sid, pl.ds(g * SEG, SEG), :],
                                gidx_v)
                pltpu.sync_copy(sidx.at[sid, pl.ds(g * SEG, SEG), :], sidx_v)

                def step(j, _):
                    pltpu.async_copy(table.at[gidx_v.at[j]], rows_v,
                                     sem).wait()
                    pltpu.sync_copy(rows_v, acc.at[sidx_v.at[j]], add=True)
                    return 0

                lax.fori_loop(0, SEG, step, 0)
                return 0

            lax.fori_loop(0, nseg, seg_step, 0)
            copy_out(acc, c * LANE)

        def run_counts(acc):
            acc_zero(acc)

            def seg_step(g, _):
                pltpu.sync_copy(sidx.at[sid, pl.ds(g * SEG, SEG), :], sidx_v)

                def step(j, _):
                    pltpu.sync_copy(ones_v, acc.at[sidx_v.at[j]], add=True)
                    return 0

                lax.fori_loop(0, SEG, step, 0)
                return 0

            lax.fori_loop(0, nseg, seg_step, 0)
            copy_out(acc, NCH * LANE)

        @pl.when(cid == 0)
        def _():
            for c in range(NCH // 2):
                run_chunk(acc, c)
            if with_counts:
                run_counts(acc)

        @pl.when(cid == 1)
        def _():
            for c in range(NCH // 2, NCH):
                run_chunk(acc, c)

    return pl.kernel(
        body,
        out_type=jax.ShapeDtypeStruct((n_nodes, R, cols), jnp.float32),
        mesh=mesh,
        scratch_types=[
            pltpu.VMEM((SEG, BW), jnp.int32),
            pltpu.VMEM((SEG, BW), jnp.int32),
            pltpu.VMEM((BW, LANE), jnp.float32),
            pltpu.VMEM((zr, LANE), jnp.float32),
            pltpu.VMEM((BW, LANE), jnp.float32),
            pltpu.VMEM_SHARED((acc_rows, LANE), jnp.float32),
            pltpu.SemaphoreType.DMA,
        ],
        compiler_params=pltpu.CompilerParams(use_tc_tiling_on_sc=False),
    )


def _tc_layer1(x, agg, root, W, b, bn=400):
    n = x.shape[0]

    def body(x_ref, agg_ref, root_ref, w_ref, b_ref, out_ref):
        acc = jnp.dot(x_ref[...], root_ref[...],
                      preferred_element_type=jnp.float32) + b_ref[...]
        for r in range(R):
            s = agg_ref[:, r, 0:128]
            cnt = agg_ref[:, r, 128:129]
            h = s / jnp.maximum(cnt, 1.0)
            acc = acc + jnp.dot(h, w_ref[r], preferred_element_type=jnp.float32)
        out_ref[...] = jnp.maximum(acc, 0.0)

    return pl.pallas_call(
        body,
        grid=(n // bn,),
        in_specs=[
            pl.BlockSpec((bn, 128), lambda i: (i, 0)),
            pl.BlockSpec((bn, R, 144), lambda i: (i, 0, 0)),
            pl.BlockSpec((128, 128), lambda i: (0, 0)),
            pl.BlockSpec((R, 128, 128), lambda i: (0, 0, 0)),
            pl.BlockSpec((1, 128), lambda i: (0, 0)),
        ],
        out_specs=pl.BlockSpec((bn, 128), lambda i: (i, 0)),
        out_shape=jax.ShapeDtypeStruct((n, 128), jnp.float32),
    )(x, agg, root, W, b)


def _tc_layer2(h, agg, cnt, root, W, b, lin_w, lin_b, bn=400):
    n = h.shape[0]
    d_out = lin_w.shape[1]

    def body(h_ref, agg_ref, cnt_ref, root_ref, w_ref, b_ref, lw_ref, lb_ref,
             out_ref):
        acc = jnp.dot(h_ref[...], root_ref[...],
                      preferred_element_type=jnp.float32) + b_ref[...]
        for r in range(R):
            s = agg_ref[:, r, :]
            c = cnt_ref[:, r:r + 1]
            acc = acc + jnp.dot(s / jnp.maximum(c, 1.0), w_ref[r],
                                preferred_element_type=jnp.float32)
        h2 = jnp.maximum(acc, 0.0)
        logits = jnp.dot(h2, lw_ref[...],
                         preferred_element_type=jnp.float32) + lb_ref[...]
        m = jnp.max(logits, axis=1, keepdims=True)
        lse = jnp.log(jnp.sum(jnp.exp(logits - m), axis=1, keepdims=True)) + m
        out_ref[...] = logits - lse

    return pl.pallas_call(
        body,
        grid=(n // bn,),
        in_specs=[
            pl.BlockSpec((bn, 128), lambda i: (i, 0)),
            pl.BlockSpec((bn, R, 128), lambda i: (i, 0, 0)),
            pl.BlockSpec((bn, R), lambda i: (i, 0)),
            pl.BlockSpec((128, 128), lambda i: (0, 0)),
            pl.BlockSpec((R, 128, 128), lambda i: (0, 0, 0)),
            pl.BlockSpec((1, 128), lambda i: (0, 0)),
            pl.BlockSpec((128, d_out), lambda i: (0, 0)),
            pl.BlockSpec((1, d_out), lambda i: (0, 0)),
        ],
        out_specs=pl.BlockSpec((bn, d_out), lambda i: (i, 0)),
        out_shape=jax.ShapeDtypeStruct((n, d_out), jnp.float32),
    )(h, agg, cnt, root, W, b, lin_w, lin_b)


def kernel(x, edge_index, edge_type, W1, root1, b1, W2, root2, b2, lin_w,
           lin_b):
    n, d_in = x.shape
    e = edge_index.shape[1]
    assert d_in == 128 and n % NTILES == 0

    nb = -(-e // (NTILES * BW * SEG)) * SEG
    e_pad = NTILES * nb * BW

    src = edge_index[0].astype(jnp.int32)
    dst = edge_index[1].astype(jnp.int32)
    et = edge_type.astype(jnp.int32)
    ar = jnp.arange(e_pad - e, dtype=jnp.int32)
    src_p = jnp.concatenate([src, ar % n])
    sidx = jnp.concatenate([et * n + dst, R * n + (ar % BW)])
    sidx = sidx.reshape(NTILES, nb, BW)
    gidx = (src_p[None, :] * NCH + jnp.arange(NCH, dtype=jnp.int32)[:, None])
    gidx = gidx.reshape(NCH, NTILES, nb, BW)

    agg_counts = _sc_agg(True, n, nb)
    agg_plain = _sc_agg(False, n, nb)

    agg1 = agg_counts(x.reshape(n * NCH, LANE), gidx, sidx)
    h1 = _tc_layer1(x, agg1, root1, W1, b1.reshape(1, 128))
    cnt = agg1[:, :, 128]
    agg2 = agg_plain(h1.reshape(n * NCH, LANE), gidx, sidx)
    return _tc_layer2(h1, agg2, cnt, root2, W2, b2.reshape(1, 128), lin_w,
                      lin_b.reshape(1, lin_w.shape[1]))


# speedup vs baseline: 9.6192x; 1.1199x over previous
import functools

import jax
import jax.numpy as jnp
from jax import lax
from jax.experimental import pallas as pl
from jax.experimental.pallas import tpu as pltpu
from jax.experimental.pallas import tpu_sc as plsc

R = 8
LANE = 16
NTILES = 16
BW = 128
NCH = 8
SEG = 32
NBUF = 2


def _sc_agg(with_counts, n_nodes, nb):
    cols = NCH * LANE + (LANE if with_counts else 0)
    acc_rows = R * n_nodes + BW
    zshare = acc_rows // NTILES
    zr = zshare // 8
    assert zr * 8 == zshare and n_nodes % NTILES == 0
    rpt = n_nodes // NTILES
    mesh = plsc.VectorSubcoreMesh(core_axis_name="c", subcore_axis_name="s")

    nseg = nb // SEG
    assert nseg * SEG == nb

    def body(table, gidx, sidx, out, gidx_v, sidx_v, rows_v, zbuf, ones_v,
             acc, sem, gs0, gs1, gs2, gs3):
        gsem = [gs0, gs1, gs2, gs3]
        cid = lax.axis_index("c")
        sid = lax.axis_index("s")

        def zfill(i, _):
            zbuf[i, :] = jnp.zeros((LANE,), jnp.float32)
            return 0

        lax.fori_loop(0, zr, zfill, 0)

        if with_counts:
            def ofill(i, _):
                lane = lax.iota(jnp.int32, LANE).astype(jnp.float32)
                ones_v[i, :] = jnp.maximum(1.0 - lane, 0.0)
                return 0

            lax.fori_loop(0, BW, ofill, 0)

        def acc_zero(acc):
            for k in range(8):
                pltpu.sync_copy(zbuf, acc.at[pl.ds(sid * zshare + k * zr, zr), :])
            plsc.subcore_barrier()

        def copy_out(acc, c0):
            plsc.subcore_barrier()
            for r in range(R):
                pltpu.sync_copy(
                    acc.at[pl.ds(r * n_nodes + sid * rpt, rpt), :],
                    out.at[pl.ds(sid * rpt, rpt), r, pl.ds(c0, LANE)])
            plsc.subcore_barrier()

        def run_chunk(acc, c):
            acc_zero(acc)

            def seg_step(g, _):
                pltpu.sync_copy(gidx.at[c, sid, pl.ds(g * SEG, SEG), :],
                                gidx_v)
                pltpu.sync_copy(sidx.at[sid, pl.ds(g * SEG, SEG), :], sidx_v)
                pend = [None] * NBUF
                pend[0] = pltpu.async_copy(
                    table.at[gidx_v.at[0]], rows_v.at[0], gsem[0])
                for i in range(SEG):
                    b = i % NBUF
                    pend[b].wait()
                    if i + 1 < SEG:
                        nb_ = (i + 1) % NBUF
                        pend[nb_] = pltpu.async_copy(
                            table.at[gidx_v.at[i + 1]], rows_v.at[nb_],
                            gsem[nb_])
                    pltpu.sync_copy(rows_v.at[b], acc.at[sidx_v.at[i]],
                                    add=True)
                return 0

            lax.fori_loop(0, nseg, seg_step, 0)
            copy_out(acc, c * LANE)

        def run_counts(acc):
            acc_zero(acc)

            def seg_step(g, _):
                pltpu.sync_copy(sidx.at[sid, pl.ds(g * SEG, SEG), :], sidx_v)

                def step(j, _):
                    pltpu.sync_copy(ones_v, acc.at[sidx_v.at[j]], add=True)
                    return 0

                lax.fori_loop(0, SEG, step, 0)
                return 0

            lax.fori_loop(0, nseg, seg_step, 0)
            copy_out(acc, NCH * LANE)

        @pl.when(cid == 0)
        def _():
            for c in range(NCH // 2):
                run_chunk(acc, c)
            if with_counts:
                run_counts(acc)

        @pl.when(cid == 1)
        def _():
            for c in range(NCH // 2, NCH):
                run_chunk(acc, c)

    return pl.kernel(
        body,
        out_type=jax.ShapeDtypeStruct((n_nodes, R, cols), jnp.float32),
        mesh=mesh,
        scratch_types=[
            pltpu.VMEM((SEG, BW), jnp.int32),
            pltpu.VMEM((SEG, BW), jnp.int32),
            pltpu.VMEM((NBUF, BW, LANE), jnp.float32),
            pltpu.VMEM((zr, LANE), jnp.float32),
            pltpu.VMEM((BW, LANE), jnp.float32),
            pltpu.VMEM_SHARED((acc_rows, LANE), jnp.float32),
            pltpu.SemaphoreType.DMA,
            pltpu.SemaphoreType.DMA,
            pltpu.SemaphoreType.DMA,
            pltpu.SemaphoreType.DMA,
            pltpu.SemaphoreType.DMA,
        ],
        compiler_params=pltpu.CompilerParams(use_tc_tiling_on_sc=False),
    )


def _tc_layer1(x, agg, root, W, b, bn=400):
    n = x.shape[0]

    def body(x_ref, agg_ref, root_ref, w_ref, b_ref, out_ref):
        acc = jnp.dot(x_ref[...], root_ref[...],
                      preferred_element_type=jnp.float32) + b_ref[...]
        for r in range(R):
            s = agg_ref[:, r, 0:128]
            cnt = agg_ref[:, r, 128:129]
            h = s / jnp.maximum(cnt, 1.0)
            acc = acc + jnp.dot(h, w_ref[r], preferred_element_type=jnp.float32)
        out_ref[...] = jnp.maximum(acc, 0.0)

    return pl.pallas_call(
        body,
        grid=(n // bn,),
        in_specs=[
            pl.BlockSpec((bn, 128), lambda i: (i, 0)),
            pl.BlockSpec((bn, R, 144), lambda i: (i, 0, 0)),
            pl.BlockSpec((128, 128), lambda i: (0, 0)),
            pl.BlockSpec((R, 128, 128), lambda i: (0, 0, 0)),
            pl.BlockSpec((1, 128), lambda i: (0, 0)),
        ],
        out_specs=pl.BlockSpec((bn, 128), lambda i: (i, 0)),
        out_shape=jax.ShapeDtypeStruct((n, 128), jnp.float32),
    )(x, agg, root, W, b)


def _tc_layer2(h, agg, cnt, root, W, b, lin_w, lin_b, bn=400):
    n = h.shape[0]
    d_out = lin_w.shape[1]

    def body(h_ref, agg_ref, cnt_ref, root_ref, w_ref, b_ref, lw_ref, lb_ref,
             out_ref):
        acc = jnp.dot(h_ref[...], root_ref[...],
                      preferred_element_type=jnp.float32) + b_ref[...]
        for r in range(R):
            s = agg_ref[:, r, :]
            c = cnt_ref[:, r:r + 1]
            acc = acc + jnp.dot(s / jnp.maximum(c, 1.0), w_ref[r],
                                preferred_element_type=jnp.float32)
        h2 = jnp.maximum(acc, 0.0)
        logits = jnp.dot(h2, lw_ref[...],
                         preferred_element_type=jnp.float32) + lb_ref[...]
        m = jnp.max(logits, axis=1, keepdims=True)
        lse = jnp.log(jnp.sum(jnp.exp(logits - m), axis=1, keepdims=True)) + m
        out_ref[...] = logits - lse

    return pl.pallas_call(
        body,
        grid=(n // bn,),
        in_specs=[
            pl.BlockSpec((bn, 128), lambda i: (i, 0)),
            pl.BlockSpec((bn, R, 128), lambda i: (i, 0, 0)),
            pl.BlockSpec((bn, R), lambda i: (i, 0)),
            pl.BlockSpec((128, 128), lambda i: (0, 0)),
            pl.BlockSpec((R, 128, 128), lambda i: (0, 0, 0)),
            pl.BlockSpec((1, 128), lambda i: (0, 0)),
            pl.BlockSpec((128, d_out), lambda i: (0, 0)),
            pl.BlockSpec((1, d_out), lambda i: (0, 0)),
        ],
        out_specs=pl.BlockSpec((bn, d_out), lambda i: (i, 0)),
        out_shape=jax.ShapeDtypeStruct((n, d_out), jnp.float32),
    )(h, agg, cnt, root, W, b, lin_w, lin_b)


def kernel(x, edge_index, edge_type, W1, root1, b1, W2, root2, b2, lin_w,
           lin_b):
    n, d_in = x.shape
    e = edge_index.shape[1]
    assert d_in == 128 and n % NTILES == 0

    nb = -(-e // (NTILES * BW * SEG)) * SEG
    e_pad = NTILES * nb * BW

    src = edge_index[0].astype(jnp.int32)
    dst = edge_index[1].astype(jnp.int32)
    et = edge_type.astype(jnp.int32)
    ar = jnp.arange(e_pad - e, dtype=jnp.int32)
    src_p = jnp.concatenate([src, ar % n])
    sidx = jnp.concatenate([et * n + dst, R * n + (ar % BW)])
    sidx = sidx.reshape(NTILES, nb, BW)
    gidx = (src_p[None, :] * NCH + jnp.arange(NCH, dtype=jnp.int32)[:, None])
    gidx = gidx.reshape(NCH, NTILES, nb, BW)

    agg_counts = _sc_agg(True, n, nb)
    agg_plain = _sc_agg(False, n, nb)

    agg1 = agg_counts(x.reshape(n * NCH, LANE), gidx, sidx)
    h1 = _tc_layer1(x, agg1, root1, W1, b1.reshape(1, 128))
    cnt = agg1[:, :, 128]
    agg2 = agg_plain(h1.reshape(n * NCH, LANE), gidx, sidx)
    return _tc_layer2(h1, agg2, cnt, root2, W2, b2.reshape(1, 128), lin_w,
                      lin_b.reshape(1, lin_w.shape[1]))


# speedup vs baseline: 16.7933x; 1.7458x over previous
import functools

import jax
import jax.numpy as jnp
from jax import lax
from jax.experimental import pallas as pl
from jax.experimental.pallas import tpu as pltpu
from jax.experimental.pallas import tpu_sc as plsc

R = 8
LANE = 16
NTILES = 16
BW = 128
NCH = 8
SEG = 32
GRP = 8


def _sc_agg(with_counts, n_nodes, nb):
    cols = NCH * LANE + (LANE if with_counts else 0)
    acc_rows = R * n_nodes + BW
    zshare = acc_rows // NTILES
    zr = zshare // 16
    assert zr * 16 == zshare and n_nodes % NTILES == 0
    rpt = n_nodes // NTILES
    mesh = plsc.VectorSubcoreMesh(core_axis_name="c", subcore_axis_name="s")

    nseg = nb // SEG
    assert nseg * SEG == nb

    ngrp = SEG // GRP

    def body(table, gidx, sidx, out, gidx_v, sidx_v, rows_v, zbuf, ones_v,
             acc, gs0, gs1, ss0, ss1):
        gsem = [gs0, gs1]
        ssem = [ss0, ss1]
        cid = lax.axis_index("c")
        sid = lax.axis_index("s")

        def zfill(i, _):
            zbuf[i, :] = jnp.zeros((LANE,), jnp.float32)
            return 0

        lax.fori_loop(0, zr, zfill, 0)

        if with_counts:
            def ofill(i, _):
                lane = lax.iota(jnp.int32, LANE).astype(jnp.float32)
                ones_v[i, :] = jnp.maximum(1.0 - lane, 0.0)
                return 0

            lax.fori_loop(0, BW, ofill, 0)

        def acc_zero(acc):
            for k in range(16):
                pltpu.sync_copy(zbuf, acc.at[pl.ds(sid * zshare + k * zr, zr), :])
            plsc.subcore_barrier()

        def copy_out(acc, c0):
            plsc.subcore_barrier()
            for r in range(R):
                pltpu.sync_copy(
                    acc.at[pl.ds(r * n_nodes + sid * rpt, rpt), :],
                    out.at[pl.ds(sid * rpt, rpt), r, pl.ds(c0, LANE)])
            plsc.subcore_barrier()

        def run_chunk(acc, c):
            acc_zero(acc)

            def seg_step(g, _):
                pltpu.sync_copy(gidx.at[c, sid, pl.ds(g * SEG, SEG), :],
                                gidx_v)
                pltpu.sync_copy(sidx.at[sid, pl.ds(g * SEG, SEG), :], sidx_v)
                gd = [[None] * GRP for _ in range(2)]
                sd = [[None] * GRP for _ in range(2)]

                def fire_gathers(grp):
                    p = grp % 2
                    for k in range(GRP):
                        gd[p][k] = pltpu.async_copy(
                            table.at[gidx_v.at[grp * GRP + k]],
                            rows_v.at[p, k], gsem[p])

                fire_gathers(0)
                for grp in range(ngrp):
                    p = grp % 2
                    if grp + 1 < ngrp:
                        if grp >= 1:
                            for k in range(GRP):
                                sd[1 - p][k].wait()
                        fire_gathers(grp + 1)
                    for k in range(GRP):
                        gd[p][k].wait()
                    for k in range(GRP):
                        sd[p][k] = pltpu.async_copy(
                            rows_v.at[p, k], acc.at[sidx_v.at[grp * GRP + k]],
                            ssem[p], add=True)
                for k in range(GRP):
                    sd[(ngrp - 2) % 2][k].wait()
                for k in range(GRP):
                    sd[(ngrp - 1) % 2][k].wait()
                return 0

            lax.fori_loop(0, nseg, seg_step, 0)
            copy_out(acc, c * LANE)

        def run_counts(acc):
            acc_zero(acc)

            def seg_step(g, _):
                pltpu.sync_copy(sidx.at[sid, pl.ds(g * SEG, SEG), :], sidx_v)

                def step(j, _):
                    pltpu.sync_copy(ones_v, acc.at[sidx_v.at[j]], add=True)
                    return 0

                lax.fori_loop(0, SEG, step, 0)
                return 0

            lax.fori_loop(0, nseg, seg_step, 0)
            copy_out(acc, NCH * LANE)

        n0 = NCH // 2 - 1 if with_counts else NCH // 2
        c_lo = jnp.where(cid == 0, 0, n0)
        c_hi = jnp.where(cid == 0, n0, NCH)

        def chunk_body(c, _):
            run_chunk(acc, c)
            return 0

        lax.fori_loop(c_lo, c_hi, chunk_body, 0)

        if with_counts:
            @pl.when(cid == 0)
            def _():
                run_counts(acc)

    return pl.kernel(
        body,
        out_type=jax.ShapeDtypeStruct((n_nodes, R, cols), jnp.float32),
        mesh=mesh,
        scratch_types=[
            pltpu.VMEM((SEG, BW), jnp.int32),
            pltpu.VMEM((SEG, BW), jnp.int32),
            pltpu.VMEM((2, GRP, BW, LANE), jnp.float32),
            pltpu.VMEM((zr, LANE), jnp.float32),
            pltpu.VMEM((BW, LANE), jnp.float32),
            pltpu.VMEM_SHARED((acc_rows, LANE), jnp.float32),
            pltpu.SemaphoreType.DMA,
            pltpu.SemaphoreType.DMA,
            pltpu.SemaphoreType.DMA,
            pltpu.SemaphoreType.DMA,
        ],
        compiler_params=pltpu.CompilerParams(use_tc_tiling_on_sc=False),
    )


def _tc_layer1(x, agg, root, W, b, bn=400):
    n = x.shape[0]

    def body(x_ref, agg_ref, root_ref, w_ref, b_ref, out_ref):
        acc = jnp.dot(x_ref[...], root_ref[...],
                      preferred_element_type=jnp.float32) + b_ref[...]
        for r in range(R):
            s = agg_ref[:, r, 0:128]
            cnt = agg_ref[:, r, 128:129]
            h = s / jnp.maximum(cnt, 1.0)
            acc = acc + jnp.dot(h, w_ref[r], preferred_element_type=jnp.float32)
        out_ref[...] = jnp.maximum(acc, 0.0)

    return pl.pallas_call(
        body,
        grid=(n // bn,),
        in_specs=[
            pl.BlockSpec((bn, 128), lambda i: (i, 0)),
            pl.BlockSpec((bn, R, 144), lambda i: (i, 0, 0)),
            pl.BlockSpec((128, 128), lambda i: (0, 0)),
            pl.BlockSpec((R, 128, 128), lambda i: (0, 0, 0)),
            pl.BlockSpec((1, 128), lambda i: (0, 0)),
        ],
        out_specs=pl.BlockSpec((bn, 128), lambda i: (i, 0)),
        out_shape=jax.ShapeDtypeStruct((n, 128), jnp.float32),
    )(x, agg, root, W, b)


def _tc_layer2(h, agg, cnt, root, W, b, lin_w, lin_b, bn=400):
    n = h.shape[0]
    d_out = lin_w.shape[1]

    def body(h_ref, agg_ref, cnt_ref, root_ref, w_ref, b_ref, lw_ref, lb_ref,
             out_ref):
        acc = jnp.dot(h_ref[...], root_ref[...],
                      preferred_element_type=jnp.float32) + b_ref[...]
        for r in range(R):
            s = agg_ref[:, r, :]
            c = cnt_ref[:, r:r + 1]
            acc = acc + jnp.dot(s / jnp.maximum(c, 1.0), w_ref[r],
                                preferred_element_type=jnp.float32)
        h2 = jnp.maximum(acc, 0.0)
        logits = jnp.dot(h2, lw_ref[...],
                         preferred_element_type=jnp.float32) + lb_ref[...]
        m = jnp.max(logits, axis=1, keepdims=True)
        lse = jnp.log(jnp.sum(jnp.exp(logits - m), axis=1, keepdims=True)) + m
        out_ref[...] = logits - lse

    return pl.pallas_call(
        body,
        grid=(n // bn,),
        in_specs=[
            pl.BlockSpec((bn, 128), lambda i: (i, 0)),
            pl.BlockSpec((bn, R, 128), lambda i: (i, 0, 0)),
            pl.BlockSpec((bn, R), lambda i: (i, 0)),
            pl.BlockSpec((128, 128), lambda i: (0, 0)),
            pl.BlockSpec((R, 128, 128), lambda i: (0, 0, 0)),
            pl.BlockSpec((1, 128), lambda i: (0, 0)),
            pl.BlockSpec((128, d_out), lambda i: (0, 0)),
            pl.BlockSpec((1, d_out), lambda i: (0, 0)),
        ],
        out_specs=pl.BlockSpec((bn, d_out), lambda i: (i, 0)),
        out_shape=jax.ShapeDtypeStruct((n, d_out), jnp.float32),
    )(h, agg, cnt, root, W, b, lin_w, lin_b)


def kernel(x, edge_index, edge_type, W1, root1, b1, W2, root2, b2, lin_w,
           lin_b):
    n, d_in = x.shape
    e = edge_index.shape[1]
    assert d_in == 128 and n % NTILES == 0

    nb = -(-e // (NTILES * BW * SEG)) * SEG
    e_pad = NTILES * nb * BW

    src = edge_index[0].astype(jnp.int32)
    dst = edge_index[1].astype(jnp.int32)
    et = edge_type.astype(jnp.int32)
    ar = jnp.arange(e_pad - e, dtype=jnp.int32)
    src_p = jnp.concatenate([src, ar % n])
    sidx = jnp.concatenate([et * n + dst, R * n + (ar % BW)])
    sidx = sidx.reshape(NTILES, nb, BW)
    gidx = (src_p[None, :] * NCH + jnp.arange(NCH, dtype=jnp.int32)[:, None])
    gidx = gidx.reshape(NCH, NTILES, nb, BW)

    agg_counts = _sc_agg(True, n, nb)
    agg_plain = _sc_agg(False, n, nb)

    agg1 = agg_counts(x.reshape(n * NCH, LANE), gidx, sidx)
    h1 = _tc_layer1(x, agg1, root1, W1, b1.reshape(1, 128))
    cnt = agg1[:, :, 128]
    agg2 = agg_plain(h1.reshape(n * NCH, LANE), gidx, sidx)
    return _tc_layer2(h1, agg2, cnt, root2, W2, b2.reshape(1, 128), lin_w,
                      lin_b.reshape(1, lin_w.shape[1]))


# speedup vs baseline: 16.9445x; 1.0090x over previous
import functools

import jax
import jax.numpy as jnp
from jax import lax
from jax.experimental import pallas as pl
from jax.experimental.pallas import tpu as pltpu
from jax.experimental.pallas import tpu_sc as plsc

R = 8
LANE = 16
NTILES = 16
BW = 128
NCH = 8
SEG = 32
GRP = 8


def _sc_agg(with_counts, n_nodes, nb):
    cols = NCH * LANE + (LANE if with_counts else 0)
    acc_rows = R * n_nodes + BW
    zshare = acc_rows // NTILES
    zr = zshare // 16
    assert zr * 16 == zshare and n_nodes % NTILES == 0
    rpt = n_nodes // NTILES
    mesh = plsc.VectorSubcoreMesh(core_axis_name="c", subcore_axis_name="s")

    nseg = nb // SEG
    assert nseg * SEG == nb

    ngrp = SEG // GRP

    def body(table, gidx, sidx, out, gidx_v, sidx_v, rows_v, zbuf, ones_v,
             acc, gs0, gs1, ss0, ss1):
        gsem = [gs0, gs1]
        ssem = [ss0, ss1]
        cid = lax.axis_index("c")
        sid = lax.axis_index("s")

        def zfill(i, _):
            zbuf[i, :] = jnp.zeros((LANE,), jnp.float32)
            return 0

        lax.fori_loop(0, zr, zfill, 0)

        if with_counts:
            def ofill(i, _):
                lane = lax.iota(jnp.int32, LANE).astype(jnp.float32)
                ones_v[i, :] = jnp.maximum(1.0 - lane, 0.0)
                return 0

            lax.fori_loop(0, BW, ofill, 0)

        def acc_zero(acc):
            zd = [pltpu.async_copy(
                zbuf, acc.at[pl.ds(sid * zshare + k * zr, zr), :], gs0)
                for k in range(16)]
            for d in zd:
                d.wait()
            plsc.subcore_barrier()

        def copy_out(acc, c0):
            plsc.subcore_barrier()
            cd = [pltpu.async_copy(
                acc.at[pl.ds(r * n_nodes + sid * rpt, rpt), :],
                out.at[pl.ds(sid * rpt, rpt), r, pl.ds(c0, LANE)], gs0)
                for r in range(R)]
            for d in cd:
                d.wait()
            plsc.subcore_barrier()

        def run_chunk(acc, c):
            acc_zero(acc)

            def seg_step(g, _):
                pltpu.sync_copy(gidx.at[c, sid, pl.ds(g * SEG, SEG), :],
                                gidx_v)
                pltpu.sync_copy(sidx.at[sid, pl.ds(g * SEG, SEG), :], sidx_v)
                gd = [[None] * GRP for _ in range(2)]
                sd = [[None] * GRP for _ in range(2)]

                def fire_gathers(grp):
                    p = grp % 2
                    for k in range(GRP):
                        gd[p][k] = pltpu.async_copy(
                            table.at[gidx_v.at[grp * GRP + k]],
                            rows_v.at[p, k], gsem[p])

                fire_gathers(0)
                for grp in range(ngrp):
                    p = grp % 2
                    if grp + 1 < ngrp:
                        if grp >= 1:
                            for k in range(GRP):
                                sd[1 - p][k].wait()
                        fire_gathers(grp + 1)
                    for k in range(GRP):
                        gd[p][k].wait()
                    for k in range(GRP):
                        sd[p][k] = pltpu.async_copy(
                            rows_v.at[p, k], acc.at[sidx_v.at[grp * GRP + k]],
                            ssem[p], add=True)
                for k in range(GRP):
                    sd[(ngrp - 2) % 2][k].wait()
                for k in range(GRP):
                    sd[(ngrp - 1) % 2][k].wait()
                return 0

            lax.fori_loop(0, nseg, seg_step, 0)
            copy_out(acc, c * LANE)

        def run_counts(acc):
            acc_zero(acc)

            def seg_step(g, _):
                pltpu.sync_copy(sidx.at[sid, pl.ds(g * SEG, SEG), :], sidx_v)

                def step(j, _):
                    pltpu.sync_copy(ones_v, acc.at[sidx_v.at[j]], add=True)
                    return 0

                lax.fori_loop(0, SEG, step, 0)
                return 0

            lax.fori_loop(0, nseg, seg_step, 0)
            copy_out(acc, NCH * LANE)

        n0 = NCH // 2 - 1 if with_counts else NCH // 2
        c_lo = jnp.where(cid == 0, 0, n0)
        c_hi = jnp.where(cid == 0, n0, NCH)

        def chunk_body(c, _):
            run_chunk(acc, c)
            return 0

        lax.fori_loop(c_lo, c_hi, chunk_body, 0)

        if with_counts:
            @pl.when(cid == 0)
            def _():
                run_counts(acc)

    return pl.kernel(
        body,
        out_type=jax.ShapeDtypeStruct((n_nodes, R, cols), jnp.float32),
        mesh=mesh,
        scratch_types=[
            pltpu.VMEM((SEG, BW), jnp.int32),
            pltpu.VMEM((SEG, BW), jnp.int32),
            pltpu.VMEM((2, GRP, BW, LANE), jnp.float32),
            pltpu.VMEM((zr, LANE), jnp.float32),
            pltpu.VMEM((BW, LANE), jnp.float32),
            pltpu.VMEM_SHARED((acc_rows, LANE), jnp.float32),
            pltpu.SemaphoreType.DMA,
            pltpu.SemaphoreType.DMA,
            pltpu.SemaphoreType.DMA,
            pltpu.SemaphoreType.DMA,
        ],
        compiler_params=pltpu.CompilerParams(use_tc_tiling_on_sc=False),
    )


def _tc_layer1(x, agg, root, W, b, bn=400):
    n = x.shape[0]

    def body(x_ref, agg_ref, root_ref, w_ref, b_ref, out_ref):
        acc = jnp.dot(x_ref[...], root_ref[...],
                      preferred_element_type=jnp.float32) + b_ref[...]
        for r in range(R):
            s = agg_ref[:, r, 0:128]
            cnt = agg_ref[:, r, 128:129]
            h = s / jnp.maximum(cnt, 1.0)
            acc = acc + jnp.dot(h, w_ref[r], preferred_element_type=jnp.float32)
        out_ref[...] = jnp.maximum(acc, 0.0)

    return pl.pallas_call(
        body,
        grid=(n // bn,),
        in_specs=[
            pl.BlockSpec((bn, 128), lambda i: (i, 0)),
            pl.BlockSpec((bn, R, 144), lambda i: (i, 0, 0)),
            pl.BlockSpec((128, 128), lambda i: (0, 0)),
            pl.BlockSpec((R, 128, 128), lambda i: (0, 0, 0)),
            pl.BlockSpec((1, 128), lambda i: (0, 0)),
        ],
        out_specs=pl.BlockSpec((bn, 128), lambda i: (i, 0)),
        out_shape=jax.ShapeDtypeStruct((n, 128), jnp.float32),
    )(x, agg, root, W, b)


def _tc_layer2(h, agg, cnt, root, W, b, lin_w, lin_b, bn=400):
    n = h.shape[0]
    d_out = lin_w.shape[1]

    def body(h_ref, agg_ref, cnt_ref, root_ref, w_ref, b_ref, lw_ref, lb_ref,
             out_ref):
        acc = jnp.dot(h_ref[...], root_ref[...],
                      preferred_element_type=jnp.float32) + b_ref[...]
        for r in range(R):
            s = agg_ref[:, r, :]
            c = cnt_ref[:, r:r + 1]
            acc = acc + jnp.dot(s / jnp.maximum(c, 1.0), w_ref[r],
                                preferred_element_type=jnp.float32)
        h2 = jnp.maximum(acc, 0.0)
        logits = jnp.dot(h2, lw_ref[...],
                         preferred_element_type=jnp.float32) + lb_ref[...]
        m = jnp.max(logits, axis=1, keepdims=True)
        lse = jnp.log(jnp.sum(jnp.exp(logits - m), axis=1, keepdims=True)) + m
        out_ref[...] = logits - lse

    return pl.pallas_call(
        body,
        grid=(n // bn,),
        in_specs=[
            pl.BlockSpec((bn, 128), lambda i: (i, 0)),
            pl.BlockSpec((bn, R, 128), lambda i: (i, 0, 0)),
            pl.BlockSpec((bn, R), lambda i: (i, 0)),
            pl.BlockSpec((128, 128), lambda i: (0, 0)),
            pl.BlockSpec((R, 128, 128), lambda i: (0, 0, 0)),
            pl.BlockSpec((1, 128), lambda i: (0, 0)),
            pl.BlockSpec((128, d_out), lambda i: (0, 0)),
            pl.BlockSpec((1, d_out), lambda i: (0, 0)),
        ],
        out_specs=pl.BlockSpec((bn, d_out), lambda i: (i, 0)),
        out_shape=jax.ShapeDtypeStruct((n, d_out), jnp.float32),
    )(h, agg, cnt, root, W, b, lin_w, lin_b)


def kernel(x, edge_index, edge_type, W1, root1, b1, W2, root2, b2, lin_w,
           lin_b):
    n, d_in = x.shape
    e = edge_index.shape[1]
    assert d_in == 128 and n % NTILES == 0

    nb = -(-e // (NTILES * BW * SEG)) * SEG
    e_pad = NTILES * nb * BW

    src = edge_index[0].astype(jnp.int32)
    dst = edge_index[1].astype(jnp.int32)
    et = edge_type.astype(jnp.int32)
    ar = jnp.arange(e_pad - e, dtype=jnp.int32)
    src_p = jnp.concatenate([src, ar % n])
    sidx = jnp.concatenate([et * n + dst, R * n + (ar % BW)])
    sidx = sidx.reshape(NTILES, nb, BW)
    gidx = (src_p[None, :] * NCH + jnp.arange(NCH, dtype=jnp.int32)[:, None])
    gidx = gidx.reshape(NCH, NTILES, nb, BW)

    agg_counts = _sc_agg(True, n, nb)
    agg_plain = _sc_agg(False, n, nb)

    agg1 = agg_counts(x.reshape(n * NCH, LANE), gidx, sidx)
    h1 = _tc_layer1(x, agg1, root1, W1, b1.reshape(1, 128))
    cnt = agg1[:, :, 128]
    agg2 = agg_plain(h1.reshape(n * NCH, LANE), gidx, sidx)
    return _tc_layer2(h1, agg2, cnt, root2, W2, b2.reshape(1, 128), lin_w,
                      lin_b.reshape(1, lin_w.shape[1]))


# speedup vs baseline: 17.6404x; 1.0411x over previous
import functools

import jax
import jax.numpy as jnp
from jax import lax
from jax.experimental import pallas as pl
from jax.experimental.pallas import tpu as pltpu
from jax.experimental.pallas import tpu_sc as plsc

R = 8
LANE = 16
NTILES = 16
BW = 128
NCH = 8
SEG = 32
GRP = 8


def _sc_agg(with_counts, n_nodes, nb):
    acc_rows = R * n_nodes + BW
    zshare = acc_rows // NTILES
    zr = zshare // 16
    assert zr * 16 == zshare and n_nodes % NTILES == 0
    rpt = n_nodes // NTILES
    mesh = plsc.VectorSubcoreMesh(core_axis_name="c", subcore_axis_name="s")

    nseg = nb // SEG
    assert nseg * SEG == nb

    ngrp = SEG // GRP

    def body(table, gidx, sidx, *rest):
        if with_counts:
            (out, out_cnt, gidx_v, sidx_v, rows_v, zbuf, ones_v,
             acc, gs0, gs1, ss0, ss1) = rest
        else:
            (out, gidx_v, sidx_v, rows_v, zbuf, ones_v,
             acc, gs0, gs1, ss0, ss1) = rest
        gsem = [gs0, gs1]
        ssem = [ss0, ss1]
        cid = lax.axis_index("c")
        sid = lax.axis_index("s")

        def zfill(i, _):
            zbuf[i, :] = jnp.zeros((LANE,), jnp.float32)
            return 0

        lax.fori_loop(0, zr, zfill, 0)

        if with_counts:
            def ofill(i, _):
                lane = lax.iota(jnp.int32, LANE).astype(jnp.float32)
                ones_v[i, :] = jnp.maximum(1.0 - lane, 0.0)
                return 0

            lax.fori_loop(0, BW, ofill, 0)

        def acc_zero(acc):
            zd = [pltpu.async_copy(
                zbuf, acc.at[pl.ds(sid * zshare + k * zr, zr), :], gs0)
                for k in range(16)]
            for d in zd:
                d.wait()
            plsc.subcore_barrier()

        def copy_out(acc, dst, c0):
            plsc.subcore_barrier()
            cd = [pltpu.async_copy(
                acc.at[pl.ds(r * n_nodes + sid * rpt, rpt), :],
                dst.at[pl.ds(sid * rpt, rpt), r, pl.ds(c0, LANE)], gs0)
                for r in range(R)]
            for d in cd:
                d.wait()
            plsc.subcore_barrier()

        def run_chunk(acc, c):
            acc_zero(acc)

            def seg_step(g, _):
                pltpu.sync_copy(gidx.at[c, sid, pl.ds(g * SEG, SEG), :],
                                gidx_v)
                pltpu.sync_copy(sidx.at[sid, pl.ds(g * SEG, SEG), :], sidx_v)
                gd = [[None] * GRP for _ in range(2)]
                sd = [[None] * GRP for _ in range(2)]

                def fire_gathers(grp):
                    p = grp % 2
                    for k in range(GRP):
                        gd[p][k] = pltpu.async_copy(
                            table.at[gidx_v.at[grp * GRP + k]],
                            rows_v.at[p, k], gsem[p])

                fire_gathers(0)
                for grp in range(ngrp):
                    p = grp % 2
                    if grp + 1 < ngrp:
                        if grp >= 1:
                            for k in range(GRP):
                                sd[1 - p][k].wait()
                        fire_gathers(grp + 1)
                    for k in range(GRP):
                        gd[p][k].wait()
                    for k in range(GRP):
                        sd[p][k] = pltpu.async_copy(
                            rows_v.at[p, k], acc.at[sidx_v.at[grp * GRP + k]],
                            ssem[p], add=True)
                for k in range(GRP):
                    sd[(ngrp - 2) % 2][k].wait()
                for k in range(GRP):
                    sd[(ngrp - 1) % 2][k].wait()
                return 0

            lax.fori_loop(0, nseg, seg_step, 0)
            copy_out(acc, out, c * LANE)

        def run_counts(acc):
            acc_zero(acc)

            def seg_step(g, _):
                pltpu.sync_copy(sidx.at[sid, pl.ds(g * SEG, SEG), :], sidx_v)
                sd = [[None] * GRP for _ in range(2)]
                for grp in range(ngrp):
                    p = grp % 2
                    if grp >= 2:
                        for k in range(GRP):
                            sd[p][k].wait()
                    for k in range(GRP):
                        sd[p][k] = pltpu.async_copy(
                            ones_v, acc.at[sidx_v.at[grp * GRP + k]],
                            ssem[p], add=True)
                for k in range(GRP):
                    sd[(ngrp - 2) % 2][k].wait()
                for k in range(GRP):
                    sd[(ngrp - 1) % 2][k].wait()
                return 0

            lax.fori_loop(0, nseg, seg_step, 0)
            copy_out(acc, out_cnt, 0)

        n0 = NCH // 2 - 1 if with_counts else NCH // 2
        c_lo = jnp.where(cid == 0, 0, n0)
        c_hi = jnp.where(cid == 0, n0, NCH)

        def chunk_body(c, _):
            run_chunk(acc, c)
            return 0

        lax.fori_loop(c_lo, c_hi, chunk_body, 0)

        if with_counts:
            @pl.when(cid == 0)
            def _():
                run_counts(acc)

    sums_t = jax.ShapeDtypeStruct((n_nodes, R, NCH * LANE), jnp.float32)
    cnt_t = jax.ShapeDtypeStruct((n_nodes, R, LANE), jnp.float32)
    return pl.kernel(
        body,
        out_type=[sums_t, cnt_t] if with_counts else [sums_t],
        mesh=mesh,
        scratch_types=[
            pltpu.VMEM((SEG, BW), jnp.int32),
            pltpu.VMEM((SEG, BW), jnp.int32),
            pltpu.VMEM((2, GRP, BW, LANE), jnp.float32),
            pltpu.VMEM((zr, LANE), jnp.float32),
            pltpu.VMEM((BW, LANE), jnp.float32),
            pltpu.VMEM_SHARED((acc_rows, LANE), jnp.float32),
            pltpu.SemaphoreType.DMA,
            pltpu.SemaphoreType.DMA,
            pltpu.SemaphoreType.DMA,
            pltpu.SemaphoreType.DMA,
        ],
        compiler_params=pltpu.CompilerParams(use_tc_tiling_on_sc=False),
    )


def _tc_layer1(x, sums, cnt16, root, W, b, bn=400):
    n = x.shape[0]

    def body(x_ref, sums_ref, cnt_ref, root_ref, w_ref, b_ref, out_ref):
        acc = jnp.dot(x_ref[...], root_ref[...],
                      preferred_element_type=jnp.float32) + b_ref[...]
        recip = 1.0 / jnp.maximum(cnt_ref[:, :, 0], 1.0)
        for r in range(R):
            h = sums_ref[:, r, :] * recip[:, r:r + 1]
            acc = acc + jnp.dot(h, w_ref[r], preferred_element_type=jnp.float32)
        out_ref[...] = jnp.maximum(acc, 0.0)

    return pl.pallas_call(
        body,
        grid=(n // bn,),
        in_specs=[
            pl.BlockSpec((bn, 128), lambda i: (i, 0)),
            pl.BlockSpec((bn, R, 128), lambda i: (i, 0, 0)),
            pl.BlockSpec((bn, R, LANE), lambda i: (i, 0, 0)),
            pl.BlockSpec((128, 128), lambda i: (0, 0)),
            pl.BlockSpec((R, 128, 128), lambda i: (0, 0, 0)),
            pl.BlockSpec((1, 128), lambda i: (0, 0)),
        ],
        out_specs=pl.BlockSpec((bn, 128), lambda i: (i, 0)),
        out_shape=jax.ShapeDtypeStruct((n, 128), jnp.float32),
    )(x, sums, cnt16, root, W, b)


def _tc_layer2(h, sums, cnt16, root, W, b, lin_w, lin_b, bn=400):
    n = h.shape[0]
    d_out = lin_w.shape[1]

    def body(h_ref, sums_ref, cnt_ref, root_ref, w_ref, b_ref, lw_ref, lb_ref,
             out_ref):
        acc = jnp.dot(h_ref[...], root_ref[...],
                      preferred_element_type=jnp.float32) + b_ref[...]
        recip = 1.0 / jnp.maximum(cnt_ref[:, :, 0], 1.0)
        for r in range(R):
            h_r = sums_ref[:, r, :] * recip[:, r:r + 1]
            acc = acc + jnp.dot(h_r, w_ref[r],
                                preferred_element_type=jnp.float32)
        h2 = jnp.maximum(acc, 0.0)
        logits = jnp.dot(h2, lw_ref[...],
                         preferred_element_type=jnp.float32) + lb_ref[...]
        m = jnp.max(logits, axis=1, keepdims=True)
        lse = jnp.log(jnp.sum(jnp.exp(logits - m), axis=1, keepdims=True)) + m
        out_ref[...] = logits - lse

    return pl.pallas_call(
        body,
        grid=(n // bn,),
        in_specs=[
            pl.BlockSpec((bn, 128), lambda i: (i, 0)),
            pl.BlockSpec((bn, R, 128), lambda i: (i, 0, 0)),
            pl.BlockSpec((bn, R, LANE), lambda i: (i, 0, 0)),
            pl.BlockSpec((128, 128), lambda i: (0, 0)),
            pl.BlockSpec((R, 128, 128), lambda i: (0, 0, 0)),
            pl.BlockSpec((1, 128), lambda i: (0, 0)),
            pl.BlockSpec((128, d_out), lambda i: (0, 0)),
            pl.BlockSpec((1, d_out), lambda i: (0, 0)),
        ],
        out_specs=pl.BlockSpec((bn, d_out), lambda i: (i, 0)),
        out_shape=jax.ShapeDtypeStruct((n, d_out), jnp.float32),
    )(h, sums, cnt16, root, W, b, lin_w, lin_b)


def kernel(x, edge_index, edge_type, W1, root1, b1, W2, root2, b2, lin_w,
           lin_b):
    n, d_in = x.shape
    e = edge_index.shape[1]
    assert d_in == 128 and n % NTILES == 0

    nb = -(-e // (NTILES * BW * SEG)) * SEG
    e_pad = NTILES * nb * BW

    src = edge_index[0].astype(jnp.int32)
    dst = edge_index[1].astype(jnp.int32)
    et = edge_type.astype(jnp.int32)
    ar = jnp.arange(e_pad - e, dtype=jnp.int32)
    src_p = jnp.concatenate([src, ar % n])
    sidx = jnp.concatenate([et * n + dst, R * n + (ar % BW)])
    sidx = sidx.reshape(NTILES, nb, BW)
    gidx = (src_p[None, :] * NCH + jnp.arange(NCH, dtype=jnp.int32)[:, None])
    gidx = gidx.reshape(NCH, NTILES, nb, BW)

    agg_counts = _sc_agg(True, n, nb)
    agg_plain = _sc_agg(False, n, nb)

    sums1, cnt16 = agg_counts(x.reshape(n * NCH, LANE), gidx, sidx)
    h1 = _tc_layer1(x, sums1, cnt16, root1, W1, b1.reshape(1, 128))
    (sums2,) = agg_plain(h1.reshape(n * NCH, LANE), gidx, sidx)
    return _tc_layer2(h1, sums2, cnt16, root2, W2, b2.reshape(1, 128), lin_w,
                      lin_b.reshape(1, lin_w.shape[1]))


# speedup vs baseline: 18.0426x; 1.0228x over previous
import functools

import jax
import jax.numpy as jnp
from jax import lax
from jax.experimental import pallas as pl
from jax.experimental.pallas import tpu as pltpu
from jax.experimental.pallas import tpu_sc as plsc

R = 8
LANE = 16
NTILES = 16
BW = 128
NCH = 8
SEG = 32
GRP = 8


def _sc_agg(with_counts, n_nodes, nb):
    acc_rows = R * n_nodes + BW
    zshare = acc_rows // NTILES
    zr = zshare // 16
    assert zr * 16 == zshare and n_nodes % NTILES == 0
    rpt = n_nodes // NTILES
    mesh = plsc.VectorSubcoreMesh(core_axis_name="c", subcore_axis_name="s")

    nseg = nb // SEG
    assert nseg * SEG == nb

    ngrp = SEG // GRP

    def body(table, gidx, sidx, *rest):
        if with_counts:
            (out, out_cnt, gidx_v, sidx_v, rows_v, zbuf, ones_v,
             acc, gs0, gs1, ss0, ss1) = rest
        else:
            (out, gidx_v, sidx_v, rows_v, zbuf, ones_v,
             acc, gs0, gs1, ss0, ss1) = rest
        gsem = [gs0, gs1]
        ssem = [ss0, ss1]
        cid = lax.axis_index("c")
        sid = lax.axis_index("s")

        def zfill(i, _):
            zbuf[i, :] = jnp.zeros((LANE,), jnp.float32)
            return 0

        lax.fori_loop(0, zr, zfill, 0)

        if with_counts:
            def ofill(i, _):
                lane = lax.iota(jnp.int32, LANE).astype(jnp.float32)
                ones_v[i, :] = jnp.maximum(1.0 - lane, 0.0)
                return 0

            lax.fori_loop(0, BW, ofill, 0)

        def acc_zero(acc):
            zd = [pltpu.async_copy(
                zbuf, acc.at[pl.ds(sid * zshare + k * zr, zr), :], gs0)
                for k in range(16)]
            for d in zd:
                d.wait()
            plsc.subcore_barrier()

        def copy_out(acc, dst, c0):
            plsc.subcore_barrier()
            cd = [pltpu.async_copy(
                acc.at[pl.ds(r * n_nodes + sid * rpt, rpt), :],
                dst.at[pl.ds(sid * rpt, rpt), r, pl.ds(c0, LANE)], gs0)
                for r in range(R)]
            for d in cd:
                d.wait()
            plsc.subcore_barrier()

        def run_chunk(acc, c):
            acc_zero(acc)

            def seg_step(g, _):
                pltpu.sync_copy(gidx.at[c, sid, pl.ds(g * SEG, SEG), :],
                                gidx_v)
                pltpu.sync_copy(sidx.at[sid, pl.ds(g * SEG, SEG), :], sidx_v)
                gd = [[None] * GRP for _ in range(2)]
                sd = [[None] * GRP for _ in range(2)]

                def fire_gathers(grp):
                    p = grp % 2
                    for k in range(GRP):
                        gd[p][k] = pltpu.async_copy(
                            table.at[gidx_v.at[grp * GRP + k]],
                            rows_v.at[p, k], gsem[p])

                fire_gathers(0)
                for grp in range(ngrp):
                    p = grp % 2
                    if grp + 1 < ngrp:
                        if grp >= 1:
                            for k in range(GRP):
                                sd[1 - p][k].wait()
                        fire_gathers(grp + 1)
                    for k in range(GRP):
                        gd[p][k].wait()
                    for k in range(GRP):
                        sd[p][k] = pltpu.async_copy(
                            rows_v.at[p, k], acc.at[sidx_v.at[grp * GRP + k]],
                            ssem[p], add=True)
                for k in range(GRP):
                    sd[(ngrp - 2) % 2][k].wait()
                for k in range(GRP):
                    sd[(ngrp - 1) % 2][k].wait()
                return 0

            lax.fori_loop(0, nseg, seg_step, 0)
            copy_out(acc, out, c * LANE)

        def run_counts(acc):
            acc_zero(acc)

            def seg_step(g, _):
                pltpu.sync_copy(sidx.at[sid, pl.ds(g * SEG, SEG), :], sidx_v)
                sd = [[None] * GRP for _ in range(2)]
                for grp in range(ngrp):
                    p = grp % 2
                    if grp >= 2:
                        for k in range(GRP):
                            sd[p][k].wait()
                    for k in range(GRP):
                        sd[p][k] = pltpu.async_copy(
                            ones_v, acc.at[sidx_v.at[grp * GRP + k]],
                            ssem[p], add=True)
                for k in range(GRP):
                    sd[(ngrp - 2) % 2][k].wait()
                for k in range(GRP):
                    sd[(ngrp - 1) % 2][k].wait()
                return 0

            lax.fori_loop(0, nseg, seg_step, 0)
            copy_out(acc, out_cnt, 0)

        n0 = NCH // 2 - 1 if with_counts else NCH // 2
        c_lo = jnp.where(cid == 0, 0, n0)
        c_hi = jnp.where(cid == 0, n0, NCH)

        def chunk_body(c, _):
            run_chunk(acc, c)
            return 0

        lax.fori_loop(c_lo, c_hi, chunk_body, 0)

        if with_counts:
            @pl.when(cid == 0)
            def _():
                run_counts(acc)

    sums_t = jax.ShapeDtypeStruct((n_nodes, R, NCH * LANE), jnp.float32)
    cnt_t = jax.ShapeDtypeStruct((n_nodes, R, LANE), jnp.float32)
    return pl.kernel(
        body,
        out_type=[sums_t, cnt_t] if with_counts else [sums_t],
        mesh=mesh,
        scratch_types=[
            pltpu.VMEM((SEG, BW), jnp.int32),
            pltpu.VMEM((SEG, BW), jnp.int32),
            pltpu.VMEM((2, GRP, BW, LANE), jnp.float32),
            pltpu.VMEM((zr, LANE), jnp.float32),
            pltpu.VMEM((BW, LANE), jnp.float32),
            pltpu.VMEM_SHARED((acc_rows, LANE), jnp.float32),
            pltpu.SemaphoreType.DMA,
            pltpu.SemaphoreType.DMA,
            pltpu.SemaphoreType.DMA,
            pltpu.SemaphoreType.DMA,
        ],
        compiler_params=pltpu.CompilerParams(use_tc_tiling_on_sc=False),
    )


def _bcast_col(recip, r):
    sel = (lax.broadcasted_iota(jnp.int32, (R, 128), 0) == r)
    return jnp.dot(recip, sel.astype(jnp.float32),
                   preferred_element_type=jnp.float32)


def _tc_layer1(x, sums, cnt8, root, W, b, bn=400):
    n = x.shape[0]

    def body(x_ref, sums_ref, cnt_ref, root_ref, w_ref, b_ref, out_ref):
        acc = jnp.dot(x_ref[...], root_ref[...],
                      preferred_element_type=jnp.float32) + b_ref[...]
        recip = 1.0 / jnp.maximum(cnt_ref[...], 1.0)
        for r in range(R):
            h = sums_ref[:, r, :] * _bcast_col(recip, r)
            acc = acc + jnp.dot(h, w_ref[r], preferred_element_type=jnp.float32)
        out_ref[...] = jnp.maximum(acc, 0.0)

    return pl.pallas_call(
        body,
        grid=(n // bn,),
        in_specs=[
            pl.BlockSpec((bn, 128), lambda i: (i, 0)),
            pl.BlockSpec((bn, R, 128), lambda i: (i, 0, 0)),
            pl.BlockSpec((bn, R), lambda i: (i, 0)),
            pl.BlockSpec((128, 128), lambda i: (0, 0)),
            pl.BlockSpec((R, 128, 128), lambda i: (0, 0, 0)),
            pl.BlockSpec((1, 128), lambda i: (0, 0)),
        ],
        out_specs=pl.BlockSpec((bn, 128), lambda i: (i, 0)),
        out_shape=jax.ShapeDtypeStruct((n, 128), jnp.float32),
    )(x, sums, cnt8, root, W, b)


def _tc_layer2(h, sums, cnt8, root, W, b, lin_w, lin_b, bn=400):
    n = h.shape[0]
    d_out = lin_w.shape[1]

    def body(h_ref, sums_ref, cnt_ref, root_ref, w_ref, b_ref, lw_ref, lb_ref,
             out_ref):
        acc = jnp.dot(h_ref[...], root_ref[...],
                      preferred_element_type=jnp.float32) + b_ref[...]
        recip = 1.0 / jnp.maximum(cnt_ref[...], 1.0)
        for r in range(R):
            h_r = sums_ref[:, r, :] * _bcast_col(recip, r)
            acc = acc + jnp.dot(h_r, w_ref[r],
                                preferred_element_type=jnp.float32)
        h2 = jnp.maximum(acc, 0.0)
        logits = jnp.dot(h2, lw_ref[...],
                         preferred_element_type=jnp.float32) + lb_ref[...]
        m = jnp.max(logits, axis=1, keepdims=True)
        lse = jnp.log(jnp.sum(jnp.exp(logits - m), axis=1, keepdims=True)) + m
        out_ref[...] = logits - lse

    return pl.pallas_call(
        body,
        grid=(n // bn,),
        in_specs=[
            pl.BlockSpec((bn, 128), lambda i: (i, 0)),
            pl.BlockSpec((bn, R, 128), lambda i: (i, 0, 0)),
            pl.BlockSpec((bn, R), lambda i: (i, 0)),
            pl.BlockSpec((128, 128), lambda i: (0, 0)),
            pl.BlockSpec((R, 128, 128), lambda i: (0, 0, 0)),
            pl.BlockSpec((1, 128), lambda i: (0, 0)),
            pl.BlockSpec((128, d_out), lambda i: (0, 0)),
            pl.BlockSpec((1, d_out), lambda i: (0, 0)),
        ],
        out_specs=pl.BlockSpec((bn, d_out), lambda i: (i, 0)),
        out_shape=jax.ShapeDtypeStruct((n, d_out), jnp.float32),
    )(h, sums, cnt8, root, W, b, lin_w, lin_b)


def kernel(x, edge_index, edge_type, W1, root1, b1, W2, root2, b2, lin_w,
           lin_b):
    n, d_in = x.shape
    e = edge_index.shape[1]
    assert d_in == 128 and n % NTILES == 0

    nb = -(-e // (NTILES * BW * SEG)) * SEG
    e_pad = NTILES * nb * BW

    src = edge_index[0].astype(jnp.int32)
    dst = edge_index[1].astype(jnp.int32)
    et = edge_type.astype(jnp.int32)
    ar = jnp.arange(e_pad - e, dtype=jnp.int32)
    src_p = jnp.concatenate([src, ar % n])
    sidx = jnp.concatenate([et * n + dst, R * n + (ar % BW)])
    sidx = sidx.reshape(NTILES, nb, BW)
    gidx = (src_p[None, :] * NCH + jnp.arange(NCH, dtype=jnp.int32)[:, None])
    gidx = gidx.reshape(NCH, NTILES, nb, BW)

    agg_counts = _sc_agg(True, n, nb)
    agg_plain = _sc_agg(False, n, nb)

    sums1, cnt16 = agg_counts(x.reshape(n * NCH, LANE), gidx, sidx)
    cnt8 = cnt16[:, :, 0]
    h1 = _tc_layer1(x, sums1, cnt8, root1, W1, b1.reshape(1, 128))
    (sums2,) = agg_plain(h1.reshape(n * NCH, LANE), gidx, sidx)
    return _tc_layer2(h1, sums2, cnt8, root2, W2, b2.reshape(1, 128), lin_w,
                      lin_b.reshape(1, lin_w.shape[1]))
